# Initial kernel scaffold; baseline (speedup 1.0000x reference)
import jax
import jax.numpy as jnp
from jax.experimental import pallas as pl


def kernel(attn_bias, spatial_pos, x, edge_input, attn_edge_type, edge_encoder_weight, spatial_pos_encoder_weight, graph_token_virtual_distance_weight):
    raise NotImplementedError("write your pallas kernel here")


# speedup vs baseline: 9.0102x; 9.0102x over previous
import functools

import jax
import jax.numpy as jnp
from jax import lax
from jax.experimental import pallas as pl
from jax.experimental.pallas import tpu as pltpu
from jax.experimental.pallas import tpu_sc as plsc

B, N, H = 64, 128, 32
NP1 = N + 1
NUM_ITEMS = B * N
NUM_WORKERS = 32
ITEMS_PER_WORKER = NUM_ITEMS // NUM_WORKERS


def _sc_gather_body(spat_tab_1d, edge_tab_1d, sp_idx_hbm, aet_idx_hbm,
                    inner_hbm, sp_idx_v, aet_idx_v, spat_rows, edge_rows,
                    out_buf):
    spat_tab = spat_tab_1d
    edge_tab = edge_tab_1d
    wid = lax.axis_index("s") * 2 + lax.axis_index("c")
    base = wid * ITEMS_PER_WORKER

    def per_item(it, carry):
        item = base + it
        pltpu.sync_copy(sp_idx_hbm.at[item], sp_idx_v)
        pltpu.sync_copy(aet_idx_hbm.at[item], aet_idx_v)
        pltpu.sync_copy(spat_tab.at[sp_idx_v], spat_rows)
        for r in range(3):
            pltpu.sync_copy(edge_tab.at[aet_idx_v.at[r]],
                            edge_rows.at[pl.ds(r * N, N)])

        def per_cell(c, carry2):
            lo = pl.ds(0, 16)
            hi = pl.ds(16, 16)
            s_lo = spat_rows[c, lo] + edge_rows[3 * c, lo]
            s_hi = spat_rows[c, hi] + edge_rows[3 * c, hi]
            s_lo = s_lo + (edge_rows[3 * c + 1, lo] + edge_rows[3 * c + 2, lo])
            s_hi = s_hi + (edge_rows[3 * c + 1, hi] + edge_rows[3 * c + 2, hi])
            out_buf[c, lo] = s_lo
            out_buf[c, hi] = s_hi
            return carry2

        lax.fori_loop(0, N, per_cell, 0, unroll=8)
        pltpu.sync_copy(out_buf, inner_hbm.at[pl.ds(item * N, N)])
        return carry

    lax.fori_loop(0, ITEMS_PER_WORKER, per_item, 0)


def _sc_gather(spat_tab_bf16, edge_tab_bf16, sp_idx, aet_idx):
    mesh = plsc.VectorSubcoreMesh(core_axis_name="c", subcore_axis_name="s")
    return pl.kernel(
        _sc_gather_body,
        out_type=jax.ShapeDtypeStruct((B * N * N, H), jnp.float32),
        mesh=mesh,
        scratch_types=[
            pltpu.VMEM((N,), jnp.int32),
            pltpu.VMEM((3, N), jnp.int32),
            pltpu.VMEM((N, H), jnp.float32),
            pltpu.VMEM((3 * N, H), jnp.float32),
            pltpu.VMEM((N, H), jnp.float32),
        ],
        compiler_params=pltpu.CompilerParams(use_tc_tiling_on_sc=False),
    )(spat_tab_bf16, edge_tab_bf16, sp_idx, aet_idx)


def _tc_assemble_body(ab_ref, inner_ref, t_ref, out_ref):
    tv = t_ref[0]
    ab_row0 = ab_ref[0, 0, :] * 2.0
    out_ref[0, :, 0, :] = ab_row0[None, :] + tv[:, None]

    def per_row(i, carry):
        tile = inner_ref[0, i]
        tile_t = jnp.transpose(tile)
        row_bias = jnp.concatenate([tv[:, None], tile_t], axis=1)
        ab_row = ab_ref[0, i + 1, :] * 2.0
        out_ref[0, :, i + 1, :] = ab_row[None, :] + row_bias
        return carry

    lax.fori_loop(0, N, per_row, 0)


def _tc_assemble(attn_bias, inner, t):
    return pl.pallas_call(
        _tc_assemble_body,
        grid=(B,),
        in_specs=[
            pl.BlockSpec((1, NP1, NP1), lambda b: (b, 0, 0)),
            pl.BlockSpec((1, N, N, H), lambda b: (b, 0, 0, 0)),
            pl.BlockSpec((1, H), lambda b: (0, 0)),
        ],
        out_specs=pl.BlockSpec((1, H, NP1, NP1), lambda b: (b, 0, 0, 0)),
        out_shape=jax.ShapeDtypeStruct((B, H, NP1, NP1), jnp.float32),
    )(attn_bias, inner, t)


@jax.jit
def _run(attn_bias, spatial_pos, attn_edge_type,
         edge_encoder_weight, spatial_pos_encoder_weight,
         graph_token_virtual_distance_weight):
    spat_tab = spatial_pos_encoder_weight
    edge_tab = edge_encoder_weight * (1.0 / 3.0)
    sp_idx = spatial_pos.reshape(NUM_ITEMS, N)
    aet_idx = attn_edge_type.reshape(NUM_ITEMS, 3, N)
    inner = _sc_gather(spat_tab, edge_tab, sp_idx, aet_idx)
    inner4 = inner.reshape(B, N, N, H)
    return _tc_assemble(attn_bias, inner4,
                        graph_token_virtual_distance_weight)


def kernel(attn_bias, spatial_pos, x, edge_input, attn_edge_type,
           edge_encoder_weight, spatial_pos_encoder_weight,
           graph_token_virtual_distance_weight):
    del x, edge_input
    return _run(attn_bias, spatial_pos, attn_edge_type,
                edge_encoder_weight, spatial_pos_encoder_weight,
                graph_token_virtual_distance_weight)


# speedup vs baseline: 11.7657x; 1.3058x over previous
import functools

import jax
import jax.numpy as jnp
from jax import lax
from jax.experimental import pallas as pl
from jax.experimental.pallas import tpu as pltpu
from jax.experimental.pallas import tpu_sc as plsc

B, N, H = 64, 128, 32
NP1 = N + 1
NUM_ITEMS = B * N
NUM_WORKERS = 32
ITEMS_PER_WORKER = NUM_ITEMS // NUM_WORKERS


def _sc_gather_body(spat_tab, edge_tab, sp_idx_hbm, aet_idx_hbm,
                    inner_hbm,
                    sp_idx0, sp_idx1, aet_idx0, aet_idx1,
                    spat_rows0, spat_rows1, edge_rows0, edge_rows1,
                    out_buf0, out_buf1,
                    sem_i0, sem_i1, sem_g0, sem_g1, sem_o0, sem_o1):
    wid = lax.axis_index("s") * 2 + lax.axis_index("c")
    base = wid * ITEMS_PER_WORKER
    n = ITEMS_PER_WORKER
    sp_idx = (sp_idx0, sp_idx1)
    aet_idx = (aet_idx0, aet_idx1)
    spat_rows = (spat_rows0, spat_rows1)
    edge_rows = (edge_rows0, edge_rows1)
    out_buf = (out_buf0, out_buf1)
    sem_i = (sem_i0, sem_i1)
    sem_g = (sem_g0, sem_g1)
    sem_o = (sem_o0, sem_o1)

    def start_idx(item, p):
        pltpu.async_copy(sp_idx_hbm.at[item], sp_idx[p], sem_i[p])
        pltpu.async_copy(aet_idx_hbm.at[item], aet_idx[p], sem_i[p])

    def wait_idx(p):
        pltpu.make_async_copy(sp_idx_hbm.at[0], sp_idx[p], sem_i[p]).wait()
        pltpu.make_async_copy(aet_idx_hbm.at[0], aet_idx[p], sem_i[p]).wait()

    def start_gathers(p):
        pltpu.async_copy(spat_tab.at[sp_idx[p]], spat_rows[p], sem_g[p])
        for r in range(3):
            pltpu.async_copy(edge_tab.at[aet_idx[p].at[r]],
                             edge_rows[p].at[pl.ds(r * N, N)], sem_g[p])

    def wait_gathers(p):
        pltpu.make_async_copy(
            spat_tab.at[sp_idx[p]], spat_rows[p], sem_g[p]).wait()
        for r in range(3):
            pltpu.make_async_copy(
                edge_tab.at[aet_idx[p].at[r]],
                edge_rows[p].at[pl.ds(r * N, N)], sem_g[p]).wait()

    def start_out(item, p):
        pltpu.async_copy(out_buf[p], inner_hbm.at[pl.ds(item * N, N)],
                         sem_o[p])

    def wait_out(p):
        pltpu.make_async_copy(
            out_buf[p], inner_hbm.at[pl.ds(0, N)], sem_o[p]).wait()

    pltpu.sync_copy(sp_idx_hbm.at[base], sp_idx[0])
    pltpu.sync_copy(aet_idx_hbm.at[base], aet_idx[0])
    start_gathers(0)
    start_idx(base + 1, 1)

    def per_pair(it2, carry):
        for p in (0, 1):
            q = 1 - p
            it = 2 * it2 + p
            item = base + it
            wait_gathers(p)

            @pl.when(it + 2 < n)
            def _():
                start_idx(item + 2, p)

            @pl.when(it + 1 < n)
            def _():
                wait_idx(q)
                start_gathers(q)

            @pl.when(it >= 2)
            def _():
                wait_out(p)

            def per_cell(c, carry2):
                lo = pl.ds(0, 16)
                hi = pl.ds(16, 16)
                sr, er, ob = spat_rows[p], edge_rows[p], out_buf[p]
                s_lo = sr[c, lo] + er[3 * c, lo]
                s_hi = sr[c, hi] + er[3 * c, hi]
                s_lo = s_lo + (er[3 * c + 1, lo] + er[3 * c + 2, lo])
                s_hi = s_hi + (er[3 * c + 1, hi] + er[3 * c + 2, hi])
                ob[c, lo] = s_lo
                ob[c, hi] = s_hi
                return carry2

            lax.fori_loop(0, N, per_cell, 0, unroll=8)
            start_out(item, p)
        return carry

    lax.fori_loop(0, n // 2, per_pair, 0)
    wait_out(0)
    wait_out(1)


def _sc_gather(spat_tab_f32, edge_tab_f32, sp_idx, aet_idx):
    mesh = plsc.VectorSubcoreMesh(core_axis_name="c", subcore_axis_name="s")
    return pl.kernel(
        _sc_gather_body,
        out_type=jax.ShapeDtypeStruct((B * N * N, H), jnp.float32),
        mesh=mesh,
        scratch_types=[
            pltpu.VMEM((N,), jnp.int32),
            pltpu.VMEM((N,), jnp.int32),
            pltpu.VMEM((3, N), jnp.int32),
            pltpu.VMEM((3, N), jnp.int32),
            pltpu.VMEM((N, H), jnp.float32),
            pltpu.VMEM((N, H), jnp.float32),
            pltpu.VMEM((3 * N, H), jnp.float32),
            pltpu.VMEM((3 * N, H), jnp.float32),
            pltpu.VMEM((N, H), jnp.float32),
            pltpu.VMEM((N, H), jnp.float32),
            pltpu.SemaphoreType.DMA,
            pltpu.SemaphoreType.DMA,
            pltpu.SemaphoreType.DMA,
            pltpu.SemaphoreType.DMA,
            pltpu.SemaphoreType.DMA,
            pltpu.SemaphoreType.DMA,
        ],
        compiler_params=pltpu.CompilerParams(use_tc_tiling_on_sc=False),
    )(spat_tab_f32, edge_tab_f32, sp_idx, aet_idx)


def _tc_assemble_body(ab_ref, inner_ref, t_ref, out_ref):
    tv = t_ref[0]
    ab_row0 = ab_ref[0, 0, :] * 2.0
    out_ref[0, :, 0, :] = ab_row0[None, :] + tv[:, None]

    def per_row(i, carry):
        tile = inner_ref[0, i]
        tile_t = jnp.transpose(tile)
        row_bias = jnp.concatenate([tv[:, None], tile_t], axis=1)
        ab_row = ab_ref[0, i + 1, :] * 2.0
        out_ref[0, :, i + 1, :] = ab_row[None, :] + row_bias
        return carry

    lax.fori_loop(0, N, per_row, 0)


def _tc_assemble(attn_bias, inner, t):
    return pl.pallas_call(
        _tc_assemble_body,
        grid=(B,),
        in_specs=[
            pl.BlockSpec((1, NP1, NP1), lambda b: (b, 0, 0)),
            pl.BlockSpec((1, N, N, H), lambda b: (b, 0, 0, 0)),
            pl.BlockSpec((1, H), lambda b: (0, 0)),
        ],
        out_specs=pl.BlockSpec((1, H, NP1, NP1), lambda b: (b, 0, 0, 0)),
        out_shape=jax.ShapeDtypeStruct((B, H, NP1, NP1), jnp.float32),
    )(attn_bias, inner, t)


@jax.jit
def _run(attn_bias, spatial_pos, attn_edge_type,
         edge_encoder_weight, spatial_pos_encoder_weight,
         graph_token_virtual_distance_weight):
    spat_tab = spatial_pos_encoder_weight
    edge_tab = edge_encoder_weight * (1.0 / 3.0)
    sp_idx = spatial_pos.reshape(NUM_ITEMS, N)
    aet_idx = attn_edge_type.reshape(NUM_ITEMS, 3, N)
    inner = _sc_gather(spat_tab, edge_tab, sp_idx, aet_idx)
    inner4 = inner.reshape(B, N, N, H)
    return _tc_assemble(attn_bias, inner4,
                        graph_token_virtual_distance_weight)


def kernel(attn_bias, spatial_pos, x, edge_input, attn_edge_type,
           edge_encoder_weight, spatial_pos_encoder_weight,
           graph_token_virtual_distance_weight):
    del x, edge_input
    return _run(attn_bias, spatial_pos, attn_edge_type,
                edge_encoder_weight, spatial_pos_encoder_weight,
                graph_token_virtual_distance_weight)


# speedup vs baseline: 18.3724x; 1.5615x over previous
import functools

import jax
import jax.numpy as jnp
from jax import lax
from jax.experimental import pallas as pl
from jax.experimental.pallas import tpu as pltpu
from jax.experimental.pallas import tpu_sc as plsc

B, N, H = 64, 128, 32
NP1 = N + 1
NUM_ITEMS = B * N
NUM_WORKERS = 32
ITEMS_PER_WORKER = NUM_ITEMS // NUM_WORKERS


def _sc_gather_body(spat_tab, edge_tab, sp_idx_hbm, aet_idx_hbm,
                    inner_hbm,
                    sp_idx0, sp_idx1, aet_idx0, aet_idx1,
                    spat_rows0, spat_rows1, edge_rows0, edge_rows1,
                    out_buf0, out_buf1,
                    sem_i0, sem_i1, sem_g0, sem_g1, sem_o0, sem_o1):
    wid = lax.axis_index("s") * 2 + lax.axis_index("c")
    base = wid * ITEMS_PER_WORKER
    n = ITEMS_PER_WORKER
    sp_idx = (sp_idx0, sp_idx1)
    aet_idx = (aet_idx0, aet_idx1)
    spat_rows = (spat_rows0, spat_rows1)
    edge_rows = (edge_rows0, edge_rows1)
    out_buf = (out_buf0, out_buf1)
    sem_i = (sem_i0, sem_i1)
    sem_g = (sem_g0, sem_g1)
    sem_o = (sem_o0, sem_o1)

    def start_idx(item, p):
        pltpu.async_copy(sp_idx_hbm.at[item], sp_idx[p], sem_i[p])
        pltpu.async_copy(aet_idx_hbm.at[item], aet_idx[p], sem_i[p])

    def wait_idx(p):
        pltpu.make_async_copy(sp_idx_hbm.at[0], sp_idx[p], sem_i[p]).wait()
        pltpu.make_async_copy(aet_idx_hbm.at[0], aet_idx[p], sem_i[p]).wait()

    def start_gathers(p):
        pltpu.async_copy(spat_tab.at[sp_idx[p]], spat_rows[p], sem_g[p])
        for r in range(3):
            pltpu.async_copy(edge_tab.at[aet_idx[p].at[r]],
                             edge_rows[p].at[pl.ds(r * N, N)], sem_g[p])

    def wait_gathers(p):
        pltpu.make_async_copy(
            spat_tab.at[sp_idx[p]], spat_rows[p], sem_g[p]).wait()
        for r in range(3):
            pltpu.make_async_copy(
                edge_tab.at[aet_idx[p].at[r]],
                edge_rows[p].at[pl.ds(r * N, N)], sem_g[p]).wait()

    def start_out(item, p):
        pltpu.async_copy(out_buf[p], inner_hbm.at[pl.ds(item * N, N)],
                         sem_o[p])

    def wait_out(p):
        pltpu.make_async_copy(
            out_buf[p], inner_hbm.at[pl.ds(0, N)], sem_o[p]).wait()

    pltpu.sync_copy(sp_idx_hbm.at[base], sp_idx[0])
    pltpu.sync_copy(aet_idx_hbm.at[base], aet_idx[0])
    start_gathers(0)
    start_idx(base + 1, 1)

    def per_pair(it2, carry):
        for p in (0, 1):
            q = 1 - p
            it = 2 * it2 + p
            item = base + it
            wait_gathers(p)

            @pl.when(it + 2 < n)
            def _():
                start_idx(item + 2, p)

            @pl.when(it + 1 < n)
            def _():
                wait_idx(q)
                start_gathers(q)

            @pl.when(it >= 2)
            def _():
                wait_out(p)

            def per_cell(c, carry2):
                lo = pl.ds(0, 16)
                hi = pl.ds(16, 16)
                sr, er, ob = spat_rows[p], edge_rows[p], out_buf[p]
                s_lo = sr[c, lo] + er[3 * c, lo]
                s_hi = sr[c, hi] + er[3 * c, hi]
                s_lo = s_lo + (er[3 * c + 1, lo] + er[3 * c + 2, lo])
                s_hi = s_hi + (er[3 * c + 1, hi] + er[3 * c + 2, hi])
                ob[c, lo] = s_lo
                ob[c, hi] = s_hi
                return carry2

            lax.fori_loop(0, N, per_cell, 0, unroll=8)
            start_out(item, p)
        return carry

    lax.fori_loop(0, n // 2, per_pair, 0)
    wait_out(0)
    wait_out(1)


def _sc_gather(spat_tab_f32, edge_tab_f32, sp_idx, aet_idx):
    mesh = plsc.VectorSubcoreMesh(core_axis_name="c", subcore_axis_name="s")
    return pl.kernel(
        _sc_gather_body,
        out_type=jax.ShapeDtypeStruct((B * N * N, H), jnp.float32),
        mesh=mesh,
        scratch_types=[
            pltpu.VMEM((N,), jnp.int32),
            pltpu.VMEM((N,), jnp.int32),
            pltpu.VMEM((3, N), jnp.int32),
            pltpu.VMEM((3, N), jnp.int32),
            pltpu.VMEM((N, H), jnp.float32),
            pltpu.VMEM((N, H), jnp.float32),
            pltpu.VMEM((3 * N, H), jnp.float32),
            pltpu.VMEM((3 * N, H), jnp.float32),
            pltpu.VMEM((N, H), jnp.float32),
            pltpu.VMEM((N, H), jnp.float32),
            pltpu.SemaphoreType.DMA,
            pltpu.SemaphoreType.DMA,
            pltpu.SemaphoreType.DMA,
            pltpu.SemaphoreType.DMA,
            pltpu.SemaphoreType.DMA,
            pltpu.SemaphoreType.DMA,
        ],
        compiler_params=pltpu.CompilerParams(use_tc_tiling_on_sc=False),
    )(spat_tab_f32, edge_tab_f32, sp_idx, aet_idx)


def _tc_assemble_body(ab_ref, inner_ref, t_ref, out_ref):
    tv = t_ref[0]
    ab_row0 = ab_ref[0, 0, :] * 2.0
    out_ref[0, :, 0, :] = ab_row0[None, :] + tv[:, None]

    def per_row(i, carry):
        tile = inner_ref[0, i]
        tile_t = jnp.transpose(tile)
        row_bias = jnp.concatenate([tv[:, None], tile_t], axis=1)
        ab_row = ab_ref[0, i + 1, :] * 2.0
        out_ref[0, :, i + 1, :] = ab_row[None, :] + row_bias
        return carry

    lax.fori_loop(0, N, per_row, 0, unroll=8)


def _tc_assemble(attn_bias, inner, t):
    return pl.pallas_call(
        _tc_assemble_body,
        grid=(B,),
        in_specs=[
            pl.BlockSpec((1, NP1, NP1), lambda b: (b, 0, 0)),
            pl.BlockSpec((1, N, N, H), lambda b: (b, 0, 0, 0)),
            pl.BlockSpec((1, H), lambda b: (0, 0)),
        ],
        out_specs=pl.BlockSpec((1, H, NP1, NP1), lambda b: (b, 0, 0, 0)),
        out_shape=jax.ShapeDtypeStruct((B, H, NP1, NP1), jnp.float32),
    )(attn_bias, inner, t)


@jax.jit
def _run(attn_bias, spatial_pos, attn_edge_type,
         edge_encoder_weight, spatial_pos_encoder_weight,
         graph_token_virtual_distance_weight):
    spat_tab = spatial_pos_encoder_weight
    edge_tab = edge_encoder_weight * (1.0 / 3.0)
    sp_idx = spatial_pos.reshape(NUM_ITEMS, N)
    aet_idx = attn_edge_type.reshape(NUM_ITEMS, 3, N)
    inner = _sc_gather(spat_tab, edge_tab, sp_idx, aet_idx)
    inner4 = inner.reshape(B, N, N, H)
    return _tc_assemble(attn_bias, inner4,
                        graph_token_virtual_distance_weight)


def kernel(attn_bias, spatial_pos, x, edge_input, attn_edge_type,
           edge_encoder_weight, spatial_pos_encoder_weight,
           graph_token_virtual_distance_weight):
    del x, edge_input
    return _run(attn_bias, spatial_pos, attn_edge_type,
                edge_encoder_weight, spatial_pos_encoder_weight,
                graph_token_virtual_distance_weight)


# speedup vs baseline: 24.4013x; 1.3281x over previous
import functools

import jax
import jax.numpy as jnp
from jax import lax
from jax.experimental import pallas as pl
from jax.experimental.pallas import tpu as pltpu
from jax.experimental.pallas import tpu_sc as plsc

B, N, H = 64, 128, 32
NP1 = N + 1
NUM_ITEMS = B * N
NUM_WORKERS = 32
ITEMS_PER_WORKER = NUM_ITEMS // NUM_WORKERS


def _sc_gather_body(spat_tab, edge_tab, sp_idx_hbm, aet_idx_hbm,
                    inner_hbm,
                    sp_idx0, sp_idx1, aet_idx0, aet_idx1,
                    spat_rows0, spat_rows1, edge_rows0, edge_rows1,
                    out_buf0, out_buf1,
                    sem_i0, sem_i1, sem_g0, sem_g1, sem_o0, sem_o1):
    wid = lax.axis_index("s") * 2 + lax.axis_index("c")
    base = wid * ITEMS_PER_WORKER
    n = ITEMS_PER_WORKER
    sp_idx = (sp_idx0, sp_idx1)
    aet_idx = (aet_idx0, aet_idx1)
    spat_rows = (spat_rows0, spat_rows1)
    edge_rows = (edge_rows0, edge_rows1)
    out_buf = (out_buf0, out_buf1)
    sem_i = (sem_i0, sem_i1)
    sem_g = (sem_g0, sem_g1)
    sem_o = (sem_o0, sem_o1)

    def start_idx(item, p):
        pltpu.async_copy(sp_idx_hbm.at[item], sp_idx[p], sem_i[p])
        pltpu.async_copy(aet_idx_hbm.at[item], aet_idx[p], sem_i[p])

    def wait_idx(p):
        pltpu.make_async_copy(sp_idx_hbm.at[0], sp_idx[p], sem_i[p]).wait()
        pltpu.make_async_copy(aet_idx_hbm.at[0], aet_idx[p], sem_i[p]).wait()

    def start_gathers(p):
        pltpu.async_copy(spat_tab.at[sp_idx[p]], spat_rows[p], sem_g[p])
        for r in range(3):
            pltpu.async_copy(edge_tab.at[aet_idx[p].at[r]],
                             edge_rows[p].at[pl.ds(r * N, N)], sem_g[p])

    def wait_gathers(p):
        pltpu.make_async_copy(
            spat_tab.at[sp_idx[p]], spat_rows[p], sem_g[p]).wait()
        for r in range(3):
            pltpu.make_async_copy(
                edge_tab.at[aet_idx[p].at[r]],
                edge_rows[p].at[pl.ds(r * N, N)], sem_g[p]).wait()

    def start_out(it, p):
        b = 2 * wid + (it // N)
        i = it % N
        pltpu.async_copy(out_buf[p], inner_hbm.at[b, :, i, :], sem_o[p])

    def wait_out(p):
        pltpu.make_async_copy(
            out_buf[p], inner_hbm.at[0, :, 0, :], sem_o[p]).wait()

    pltpu.sync_copy(sp_idx_hbm.at[base], sp_idx[0])
    pltpu.sync_copy(aet_idx_hbm.at[base], aet_idx[0])
    start_gathers(0)
    start_idx(base + 1, 1)

    rows_lo = lax.iota(jnp.int32, 16)
    rows_hi = rows_lo + 16

    def per_pair(it2, carry):
        for p in (0, 1):
            q = 1 - p
            it = 2 * it2 + p
            item = base + it
            wait_gathers(p)

            @pl.when(it + 2 < n)
            def _():
                start_idx(item + 2, p)

            @pl.when(it + 1 < n)
            def _():
                wait_idx(q)
                start_gathers(q)

            @pl.when(it >= 2)
            def _():
                wait_out(p)

            def per_cell(c, carry2):
                lo = pl.ds(0, 16)
                hi = pl.ds(16, 16)
                sr, er, ob = spat_rows[p], edge_rows[p], out_buf[p]
                s_lo = sr[c, lo] + er[3 * c, lo]
                s_hi = sr[c, hi] + er[3 * c, hi]
                s_lo = s_lo + (er[3 * c + 1, lo] + er[3 * c + 2, lo])
                s_hi = s_hi + (er[3 * c + 1, hi] + er[3 * c + 2, hi])
                cols = jnp.broadcast_to(c, (16,)).astype(jnp.int32)
                plsc.store_scatter(ob, [rows_lo, cols], s_lo)
                plsc.store_scatter(ob, [rows_hi, cols], s_hi)
                return carry2

            lax.fori_loop(0, N, per_cell, 0, unroll=8)
            start_out(it, p)
        return carry

    lax.fori_loop(0, n // 2, per_pair, 0)
    wait_out(0)
    wait_out(1)


def _sc_gather(spat_tab_f32, edge_tab_f32, sp_idx, aet_idx):
    mesh = plsc.VectorSubcoreMesh(core_axis_name="c", subcore_axis_name="s")
    return pl.kernel(
        _sc_gather_body,
        out_type=jax.ShapeDtypeStruct((B, H, N, N), jnp.float32),
        mesh=mesh,
        scratch_types=[
            pltpu.VMEM((N,), jnp.int32),
            pltpu.VMEM((N,), jnp.int32),
            pltpu.VMEM((3, N), jnp.int32),
            pltpu.VMEM((3, N), jnp.int32),
            pltpu.VMEM((N, H), jnp.float32),
            pltpu.VMEM((N, H), jnp.float32),
            pltpu.VMEM((3 * N, H), jnp.float32),
            pltpu.VMEM((3 * N, H), jnp.float32),
            pltpu.VMEM((H, N), jnp.float32),
            pltpu.VMEM((H, N), jnp.float32),
            pltpu.SemaphoreType.DMA,
            pltpu.SemaphoreType.DMA,
            pltpu.SemaphoreType.DMA,
            pltpu.SemaphoreType.DMA,
            pltpu.SemaphoreType.DMA,
            pltpu.SemaphoreType.DMA,
        ],
        compiler_params=pltpu.CompilerParams(
            use_tc_tiling_on_sc=False, needs_layout_passes=False),
    )(spat_tab_f32, edge_tab_f32, sp_idx, aet_idx)


def _tc_assemble_body(ab_ref, mid_ref, t_ref, out_ref):
    tv = t_ref[0]
    ab_row0 = ab_ref[0, 0, :] * 2.0
    out_ref[0, :, 0, :] = ab_row0[None, :] + tv[:, None]
    ab2 = ab_ref[0, 1:, :] * 2.0
    inner_t = mid_ref[0]
    tcol = jnp.broadcast_to(tv[:, None, None], (H, N, 1))
    bias = jnp.concatenate([tcol, inner_t], axis=2)
    out_ref[0, :, 1:, :] = ab2[None, :, :] + bias


def _tc_assemble(attn_bias, mid, t):
    return pl.pallas_call(
        _tc_assemble_body,
        grid=(B,),
        in_specs=[
            pl.BlockSpec((1, NP1, NP1), lambda b: (b, 0, 0)),
            pl.BlockSpec((1, H, N, N), lambda b: (b, 0, 0, 0)),
            pl.BlockSpec((1, H), lambda b: (0, 0)),
        ],
        out_specs=pl.BlockSpec((1, H, NP1, NP1), lambda b: (b, 0, 0, 0)),
        out_shape=jax.ShapeDtypeStruct((B, H, NP1, NP1), jnp.float32),
    )(attn_bias, mid, t)


@jax.jit
def _run(attn_bias, spatial_pos, attn_edge_type,
         edge_encoder_weight, spatial_pos_encoder_weight,
         graph_token_virtual_distance_weight):
    spat_tab = spatial_pos_encoder_weight
    edge_tab = edge_encoder_weight * (1.0 / 3.0)
    sp_idx = spatial_pos.reshape(NUM_ITEMS, N)
    aet_idx = attn_edge_type.reshape(NUM_ITEMS, 3, N)
    mid = _sc_gather(spat_tab, edge_tab, sp_idx, aet_idx)
    return _tc_assemble(attn_bias, mid,
                        graph_token_virtual_distance_weight)


def kernel(attn_bias, spatial_pos, x, edge_input, attn_edge_type,
           edge_encoder_weight, spatial_pos_encoder_weight,
           graph_token_virtual_distance_weight):
    del x, edge_input
    return _run(attn_bias, spatial_pos, attn_edge_type,
                edge_encoder_weight, spatial_pos_encoder_weight,
                graph_token_virtual_distance_weight)


# speedup vs baseline: 28.7514x; 1.1783x over previous
import functools

import jax
import jax.numpy as jnp
from jax import lax
from jax.experimental import pallas as pl
from jax.experimental.pallas import tpu as pltpu
from jax.experimental.pallas import tpu_sc as plsc

B, N, H = 64, 128, 32
NP1 = N + 1
NUM_ITEMS = B * N
NUM_WORKERS = 32
ITEMS_PER_WORKER = NUM_ITEMS // NUM_WORKERS


def _sc_gather_body(spat_tab, edge_tab, sp_idx_hbm, aet_idx_hbm,
                    inner_hbm,
                    sp_idx0, sp_idx1, aet_idx0, aet_idx1,
                    spat_rows0, spat_rows1, edge_rows0, edge_rows1,
                    out_buf0, out_buf1,
                    sem_i0, sem_i1, sem_g0, sem_g1, sem_o0, sem_o1):
    wid = lax.axis_index("s") * 2 + lax.axis_index("c")
    base = wid * ITEMS_PER_WORKER
    n = ITEMS_PER_WORKER
    sp_idx = (sp_idx0, sp_idx1)
    aet_idx = (aet_idx0, aet_idx1)
    spat_rows = (spat_rows0, spat_rows1)
    edge_rows = (edge_rows0, edge_rows1)
    out_buf = (out_buf0, out_buf1)
    sem_i = (sem_i0, sem_i1)
    sem_g = (sem_g0, sem_g1)
    sem_o = (sem_o0, sem_o1)

    def start_idx(item, p):
        pltpu.async_copy(sp_idx_hbm.at[item], sp_idx[p], sem_i[p])
        pltpu.async_copy(aet_idx_hbm.at[item], aet_idx[p], sem_i[p])

    def wait_idx(p):
        pltpu.make_async_copy(sp_idx_hbm.at[0], sp_idx[p], sem_i[p]).wait()
        pltpu.make_async_copy(aet_idx_hbm.at[0], aet_idx[p], sem_i[p]).wait()

    def start_gathers(p):
        pltpu.async_copy(spat_tab.at[sp_idx[p]], spat_rows[p], sem_g[p])
        for r in range(3):
            pltpu.async_copy(edge_tab.at[aet_idx[p].at[pl.ds(r * N, N)]],
                             edge_rows[p].at[pl.ds(r * N, N)], sem_g[p])

    def wait_gathers(p):
        pltpu.make_async_copy(
            spat_tab.at[sp_idx[p]], spat_rows[p], sem_g[p]).wait()
        for r in range(3):
            pltpu.make_async_copy(
                edge_tab.at[aet_idx[p].at[pl.ds(r * N, N)]],
                edge_rows[p].at[pl.ds(r * N, N)], sem_g[p]).wait()

    def start_out(it, p):
        b = 2 * wid + (it // N)
        i = it % N
        pltpu.async_copy(out_buf[p], inner_hbm.at[b, i], sem_o[p])

    def wait_out(p):
        pltpu.make_async_copy(
            out_buf[p], inner_hbm.at[0, 0], sem_o[p]).wait()

    pltpu.sync_copy(sp_idx_hbm.at[base], sp_idx[0])
    pltpu.sync_copy(aet_idx_hbm.at[base], aet_idx[0])
    start_gathers(0)
    start_idx(base + 1, 1)

    rows_lo = lax.iota(jnp.int32, 16)
    rows_hi = rows_lo + 16

    def per_pair(it2, carry):
        for p in (0, 1):
            q = 1 - p
            it = 2 * it2 + p
            item = base + it
            wait_gathers(p)

            @pl.when(it + 2 < n)
            def _():
                start_idx(item + 2, p)

            @pl.when(it + 1 < n)
            def _():
                wait_idx(q)
                start_gathers(q)

            @pl.when(it >= 2)
            def _():
                wait_out(p)

            def per_cell(c, carry2):
                lo = pl.ds(0, 16)
                hi = pl.ds(16, 16)
                sr, er, ob = spat_rows[p], edge_rows[p], out_buf[p]
                s_lo = sr[c, lo] + er[3 * c, lo]
                s_hi = sr[c, hi] + er[3 * c, hi]
                s_lo = s_lo + (er[3 * c + 1, lo] + er[3 * c + 2, lo])
                s_hi = s_hi + (er[3 * c + 1, hi] + er[3 * c + 2, hi])
                cols = jnp.broadcast_to(c, (16,)).astype(jnp.int32)
                plsc.store_scatter(ob, [rows_lo, cols], s_lo)
                plsc.store_scatter(ob, [rows_hi, cols], s_hi)
                return carry2

            lax.fori_loop(0, N, per_cell, 0, unroll=8)
            start_out(it, p)
        return carry

    lax.fori_loop(0, n // 2, per_pair, 0)
    wait_out(0)
    wait_out(1)


def _sc_gather(spat_tab_f32, edge_tab_f32, sp_idx, aet_idx):
    mesh = plsc.VectorSubcoreMesh(core_axis_name="c", subcore_axis_name="s")
    return pl.kernel(
        _sc_gather_body,
        out_type=jax.ShapeDtypeStruct((B, N, H, N), jnp.float32),
        mesh=mesh,
        scratch_types=[
            pltpu.VMEM((N,), jnp.int32),
            pltpu.VMEM((N,), jnp.int32),
            pltpu.VMEM((3 * N,), jnp.int32),
            pltpu.VMEM((3 * N,), jnp.int32),
            pltpu.VMEM((N, H), jnp.float32),
            pltpu.VMEM((N, H), jnp.float32),
            pltpu.VMEM((3 * N, H), jnp.float32),
            pltpu.VMEM((3 * N, H), jnp.float32),
            pltpu.VMEM((H, N), jnp.float32),
            pltpu.VMEM((H, N), jnp.float32),
            pltpu.SemaphoreType.DMA,
            pltpu.SemaphoreType.DMA,
            pltpu.SemaphoreType.DMA,
            pltpu.SemaphoreType.DMA,
            pltpu.SemaphoreType.DMA,
            pltpu.SemaphoreType.DMA,
        ],
        compiler_params=pltpu.CompilerParams(
            use_tc_tiling_on_sc=False, needs_layout_passes=False),
    )(spat_tab_f32, edge_tab_f32, sp_idx, aet_idx)


def _tc_assemble_body(ab_ref, mid_ref, t_ref, out_ref):
    tv = t_ref[0]
    ab_row0 = ab_ref[0, 0, :] * 2.0
    out_ref[0, 0, :, :] = ab_row0[None, :] + tv[:, None]
    ab2 = ab_ref[0, 1:, :] * 2.0
    tcol = jnp.broadcast_to(tv[None, :, None], (N, H, 1))
    bias = jnp.concatenate([tcol, mid_ref[0]], axis=2)
    out_ref[0, 1:, :, :] = ab2[:, None, :] + bias


def _tc_assemble(attn_bias, mid, t):
    return pl.pallas_call(
        _tc_assemble_body,
        grid=(B,),
        in_specs=[
            pl.BlockSpec((1, NP1, NP1), lambda b: (b, 0, 0)),
            pl.BlockSpec((1, N, H, N), lambda b: (b, 0, 0, 0)),
            pl.BlockSpec((1, H), lambda b: (0, 0)),
        ],
        out_specs=pl.BlockSpec((1, NP1, H, NP1), lambda b: (b, 0, 0, 0)),
        out_shape=jax.ShapeDtypeStruct((B, NP1, H, NP1), jnp.float32),
    )(attn_bias, mid, t)


@jax.jit
def _run(attn_bias, spatial_pos, attn_edge_type,
         edge_encoder_weight, spatial_pos_encoder_weight,
         graph_token_virtual_distance_weight):
    spat_tab = spatial_pos_encoder_weight
    edge_tab = edge_encoder_weight * (1.0 / 3.0)
    sp_idx = spatial_pos.reshape(NUM_ITEMS, N)
    aet_idx = attn_edge_type.reshape(NUM_ITEMS, 3 * N)
    mid = _sc_gather(spat_tab, edge_tab, sp_idx, aet_idx)
    out_phys = _tc_assemble(attn_bias, mid,
                            graph_token_virtual_distance_weight)
    return jnp.transpose(out_phys, (0, 2, 1, 3))


def kernel(attn_bias, spatial_pos, x, edge_input, attn_edge_type,
           edge_encoder_weight, spatial_pos_encoder_weight,
           graph_token_virtual_distance_weight):
    del x, edge_input
    return _run(attn_bias, spatial_pos, attn_edge_type,
                edge_encoder_weight, spatial_pos_encoder_weight,
                graph_token_virtual_distance_weight)


# speedup vs baseline: 40.3621x; 1.4038x over previous
import functools

import jax
import jax.numpy as jnp
from jax import lax
from jax.experimental import pallas as pl
from jax.experimental.pallas import tpu as pltpu
from jax.experimental.pallas import tpu_sc as plsc

B, N, H = 64, 128, 32
NP1 = N + 1
NUM_ITEMS = B * N
NUM_WORKERS = 32
ITEMS_PER_WORKER = NUM_ITEMS // NUM_WORKERS


def _sc_gather_body(spat_tab, edge_tab, sp_idx_hbm, aet_idx_hbm,
                    inner_hbm,
                    sp_idx0, sp_idx1, aet_idx0, aet_idx1,
                    spat_rows0, spat_rows1, edge_rows0, edge_rows1,
                    out_buf0, out_buf1,
                    sem_i0, sem_i1, sem_g0, sem_g1, sem_o0, sem_o1):
    wid = lax.axis_index("s") * 2 + lax.axis_index("c")
    base = wid * ITEMS_PER_WORKER
    n = ITEMS_PER_WORKER
    sp_idx = (sp_idx0, sp_idx1)
    aet_idx = (aet_idx0, aet_idx1)
    spat_rows = (spat_rows0, spat_rows1)
    edge_rows = (edge_rows0, edge_rows1)
    out_buf = (out_buf0, out_buf1)
    sem_i = (sem_i0, sem_i1)
    sem_g = (sem_g0, sem_g1)
    sem_o = (sem_o0, sem_o1)

    def start_idx(item, p):
        pltpu.async_copy(sp_idx_hbm.at[item], sp_idx[p], sem_i[p])
        pltpu.async_copy(aet_idx_hbm.at[item], aet_idx[p], sem_i[p])

    def wait_idx(p):
        pltpu.make_async_copy(sp_idx_hbm.at[0], sp_idx[p], sem_i[p]).wait()
        pltpu.make_async_copy(aet_idx_hbm.at[0], aet_idx[p], sem_i[p]).wait()

    def start_gathers(p):
        pltpu.async_copy(spat_tab.at[sp_idx[p]], spat_rows[p], sem_g[p])
        for r in range(3):
            pltpu.async_copy(edge_tab.at[aet_idx[p].at[pl.ds(r * N, N)]],
                             edge_rows[p].at[pl.ds(r * N, N)], sem_g[p])

    def wait_gathers(p):
        pltpu.make_async_copy(
            spat_tab.at[sp_idx[p]], spat_rows[p], sem_g[p]).wait()
        for r in range(3):
            pltpu.make_async_copy(
                edge_tab.at[aet_idx[p].at[pl.ds(r * N, N)]],
                edge_rows[p].at[pl.ds(r * N, N)], sem_g[p]).wait()

    def start_out(it, p):
        b = 2 * wid + (it // N)
        i = it % N
        pltpu.async_copy(out_buf[p], inner_hbm.at[b, i], sem_o[p])

    def wait_out(p):
        pltpu.make_async_copy(
            out_buf[p], inner_hbm.at[0, 0], sem_o[p]).wait()

    pltpu.sync_copy(sp_idx_hbm.at[base], sp_idx[0])
    pltpu.sync_copy(aet_idx_hbm.at[base], aet_idx[0])
    start_gathers(0)
    start_idx(base + 1, 1)

    rows16 = lax.iota(jnp.int32, 16)
    cols0 = jnp.broadcast_to(jnp.int32(0), (16,))

    def per_pair(it2, carry):
        for p in (0, 1):
            q = 1 - p
            it = 2 * it2 + p
            item = base + it
            wait_gathers(p)

            @pl.when(it + 2 < n)
            def _():
                start_idx(item + 2, p)

            @pl.when(it + 1 < n)
            def _():
                wait_idx(q)
                start_gathers(q)

            @pl.when(it >= 2)
            def _():
                wait_out(p)

            def per_cell(c, cols):
                sr, er, ob = spat_rows[p], edge_rows[p], out_buf[p]
                ws = sr[c]
                w0 = er[3 * c]
                w1 = er[3 * c + 1]
                w2 = er[3 * c + 2]

                def f32v(w):
                    return plsc.bitcast(w, jnp.float32)

                lo = (f32v(ws << 16) + f32v(w0 << 16)) + (
                    f32v(w1 << 16) + f32v(w2 << 16))
                hi = (f32v(ws) + f32v(w0)) + (f32v(w1) + f32v(w2))
                u_lo = lax.shift_right_logical(
                    plsc.bitcast(lo, jnp.int32), 16)
                u_hi = plsc.bitcast(hi, jnp.int32) & jnp.int32(-65536)
                w_out = u_hi | u_lo
                plsc.store_scatter(ob, [rows16, cols], w_out)
                return cols + 1

            lax.fori_loop(0, N, per_cell, cols0, unroll=8)
            start_out(it, p)
        return carry

    lax.fori_loop(0, n // 2, per_pair, 0)
    wait_out(0)
    wait_out(1)


def _sc_gather(spat_tab_f32, edge_tab_f32, sp_idx, aet_idx):
    mesh = plsc.VectorSubcoreMesh(core_axis_name="c", subcore_axis_name="s")
    return pl.kernel(
        _sc_gather_body,
        out_type=jax.ShapeDtypeStruct((B, N, H // 2, N), jnp.int32),
        mesh=mesh,
        scratch_types=[
            pltpu.VMEM((N,), jnp.int32),
            pltpu.VMEM((N,), jnp.int32),
            pltpu.VMEM((3 * N,), jnp.int32),
            pltpu.VMEM((3 * N,), jnp.int32),
            pltpu.VMEM((N, H // 2), jnp.int32),
            pltpu.VMEM((N, H // 2), jnp.int32),
            pltpu.VMEM((3 * N, H // 2), jnp.int32),
            pltpu.VMEM((3 * N, H // 2), jnp.int32),
            pltpu.VMEM((H // 2, N), jnp.int32),
            pltpu.VMEM((H // 2, N), jnp.int32),
            pltpu.SemaphoreType.DMA,
            pltpu.SemaphoreType.DMA,
            pltpu.SemaphoreType.DMA,
            pltpu.SemaphoreType.DMA,
            pltpu.SemaphoreType.DMA,
            pltpu.SemaphoreType.DMA,
        ],
        compiler_params=pltpu.CompilerParams(
            use_tc_tiling_on_sc=False, needs_layout_passes=False),
    )(spat_tab_f32, edge_tab_f32, sp_idx, aet_idx)


def _tc_assemble_body(ab_ref, mid_ref, t_ref, out_ref):
    tv = t_ref[0]
    ab_row0 = ab_ref[0, 0, :] * 2.0
    out_ref[0, 0, :, :] = ab_row0[None, :] + tv[:, None]
    ab2 = ab_ref[0, 1:, :] * 2.0
    w = mid_ref[0]
    lo = lax.bitcast_convert_type(w << 16, jnp.float32)
    hi = lax.bitcast_convert_type(
        w & jnp.int32(-65536), jnp.float32)
    halves = jnp.concatenate([lo, hi], axis=1)
    tcol = jnp.broadcast_to(tv[None, :, None], (N, H, 1))
    bias = jnp.concatenate([tcol, halves], axis=2)
    out_ref[0, 1:, :, :] = ab2[:, None, :] + bias


def _tc_assemble(attn_bias, mid, t):
    return pl.pallas_call(
        _tc_assemble_body,
        grid=(B,),
        in_specs=[
            pl.BlockSpec((1, NP1, NP1), lambda b: (b, 0, 0)),
            pl.BlockSpec((1, N, H // 2, N), lambda b: (b, 0, 0, 0)),
            pl.BlockSpec((1, H), lambda b: (0, 0)),
        ],
        out_specs=pl.BlockSpec((1, NP1, H, NP1), lambda b: (b, 0, 0, 0)),
        out_shape=jax.ShapeDtypeStruct((B, NP1, H, NP1), jnp.float32),
    )(attn_bias, mid, t)


@jax.jit
def _run(attn_bias, spatial_pos, attn_edge_type,
         edge_encoder_weight, spatial_pos_encoder_weight,
         graph_token_virtual_distance_weight):
    def _pack_tab(tab_f32):
        tb = tab_f32.astype(jnp.bfloat16)
        lo = lax.bitcast_convert_type(tb[:, :16], jnp.uint16)
        hi = lax.bitcast_convert_type(tb[:, 16:], jnp.uint16)
        w = lo.astype(jnp.uint32) | (hi.astype(jnp.uint32) << 16)
        return lax.bitcast_convert_type(w, jnp.int32)

    spat_tab = _pack_tab(spatial_pos_encoder_weight)
    edge_tab = _pack_tab(edge_encoder_weight * (1.0 / 3.0))
    sp_idx = spatial_pos.reshape(NUM_ITEMS, N)
    aet_idx = attn_edge_type.reshape(NUM_ITEMS, 3 * N)
    mid = _sc_gather(spat_tab, edge_tab, sp_idx, aet_idx)
    out_phys = _tc_assemble(attn_bias, mid,
                            graph_token_virtual_distance_weight)
    return jnp.transpose(out_phys, (0, 2, 1, 3))


def kernel(attn_bias, spatial_pos, x, edge_input, attn_edge_type,
           edge_encoder_weight, spatial_pos_encoder_weight,
           graph_token_virtual_distance_weight):
    del x, edge_input
    return _run(attn_bias, spatial_pos, attn_edge_type,
                edge_encoder_weight, spatial_pos_encoder_weight,
                graph_token_virtual_distance_weight)


# speedup vs baseline: 40.4704x; 1.0027x over previous
import functools

import jax
import jax.numpy as jnp
from jax import lax
from jax.experimental import pallas as pl
from jax.experimental.pallas import tpu as pltpu
from jax.experimental.pallas import tpu_sc as plsc

B, N, H = 64, 128, 32
NP1 = N + 1
NUM_ITEMS = B * N
NUM_WORKERS = 32
ITEMS_PER_WORKER = NUM_ITEMS // NUM_WORKERS


def _sc_gather_body(spat_tab, edge_tab, sp_idx_hbm, aet_idx_hbm,
                    inner_hbm,
                    sp_idx0, sp_idx1, aet_idx0, aet_idx1,
                    spat_rows0, spat_rows1, edge_rows0, edge_rows1,
                    out_buf0, out_buf1,
                    sem_i0, sem_i1, sem_g0, sem_g1, sem_o0, sem_o1):
    wid = lax.axis_index("s") * 2 + lax.axis_index("c")
    base = wid * ITEMS_PER_WORKER
    n = ITEMS_PER_WORKER
    sp_idx = (sp_idx0, sp_idx1)
    aet_idx = (aet_idx0, aet_idx1)
    spat_rows = (spat_rows0, spat_rows1)
    edge_rows = (edge_rows0, edge_rows1)
    out_buf = (out_buf0, out_buf1)
    sem_i = (sem_i0, sem_i1)
    sem_g = (sem_g0, sem_g1)
    sem_o = (sem_o0, sem_o1)

    def start_idx(item, p):
        pltpu.async_copy(sp_idx_hbm.at[item], sp_idx[p], sem_i[p])
        pltpu.async_copy(aet_idx_hbm.at[item], aet_idx[p], sem_i[p])

    def wait_idx(p):
        pltpu.make_async_copy(sp_idx_hbm.at[0], sp_idx[p], sem_i[p]).wait()
        pltpu.make_async_copy(aet_idx_hbm.at[0], aet_idx[p], sem_i[p]).wait()

    def start_gathers(p):
        pltpu.async_copy(spat_tab.at[sp_idx[p]], spat_rows[p], sem_g[p])
        for r in range(3):
            pltpu.async_copy(edge_tab.at[aet_idx[p].at[pl.ds(r * N, N)]],
                             edge_rows[p].at[pl.ds(r * N, N)], sem_g[p])

    def wait_gathers(p):
        pltpu.make_async_copy(
            spat_tab.at[sp_idx[p]], spat_rows[p], sem_g[p]).wait()
        for r in range(3):
            pltpu.make_async_copy(
                edge_tab.at[aet_idx[p].at[pl.ds(r * N, N)]],
                edge_rows[p].at[pl.ds(r * N, N)], sem_g[p]).wait()

    def start_out(it, p):
        b = 2 * wid + (it // N)
        i = it % N
        pltpu.async_copy(out_buf[p], inner_hbm.at[b, i], sem_o[p])

    def wait_out(p):
        pltpu.make_async_copy(
            out_buf[p], inner_hbm.at[0, 0], sem_o[p]).wait()

    pltpu.sync_copy(sp_idx_hbm.at[base], sp_idx[0])
    pltpu.sync_copy(aet_idx_hbm.at[base], aet_idx[0])
    start_gathers(0)
    start_idx(base + 1, 1)

    rows16 = lax.iota(jnp.int32, 16)
    cols0 = jnp.broadcast_to(jnp.int32(0), (16,))

    def per_pair(it2, carry):
        for p in (0, 1):
            q = 1 - p
            it = 2 * it2 + p
            item = base + it
            wait_gathers(p)

            @pl.when(it + 2 < n)
            def _():
                start_idx(item + 2, p)

            @pl.when(it + 1 < n)
            def _():
                wait_idx(q)
                start_gathers(q)

            @pl.when(it >= 2)
            def _():
                wait_out(p)

            def per_cell(c, cols):
                sr, er, ob = spat_rows[p], edge_rows[p], out_buf[p]
                ws = sr[c]
                w0 = er[3 * c]
                w1 = er[3 * c + 1]
                w2 = er[3 * c + 2]

                def f32v(w):
                    return plsc.bitcast(w, jnp.float32)

                lo = (f32v(ws << 16) + f32v(w0 << 16)) + (
                    f32v(w1 << 16) + f32v(w2 << 16))
                hi = (f32v(ws) + f32v(w0)) + (f32v(w1) + f32v(w2))
                u_lo = lax.shift_right_logical(
                    plsc.bitcast(lo, jnp.int32), 16)
                u_hi = plsc.bitcast(hi, jnp.int32) & jnp.int32(-65536)
                w_out = u_hi | u_lo
                plsc.store_scatter(ob, [rows16, cols], w_out)
                return cols + 1

            lax.fori_loop(0, N, per_cell, cols0, unroll=16)
            start_out(it, p)
        return carry

    lax.fori_loop(0, n // 2, per_pair, 0)
    wait_out(0)
    wait_out(1)


def _sc_gather(spat_tab_f32, edge_tab_f32, sp_idx, aet_idx):
    mesh = plsc.VectorSubcoreMesh(core_axis_name="c", subcore_axis_name="s")
    return pl.kernel(
        _sc_gather_body,
        out_type=jax.ShapeDtypeStruct((B, N, H // 2, N), jnp.int32),
        mesh=mesh,
        scratch_types=[
            pltpu.VMEM((N,), jnp.int32),
            pltpu.VMEM((N,), jnp.int32),
            pltpu.VMEM((3 * N,), jnp.int32),
            pltpu.VMEM((3 * N,), jnp.int32),
            pltpu.VMEM((N, H // 2), jnp.int32),
            pltpu.VMEM((N, H // 2), jnp.int32),
            pltpu.VMEM((3 * N, H // 2), jnp.int32),
            pltpu.VMEM((3 * N, H // 2), jnp.int32),
            pltpu.VMEM((H // 2, N), jnp.int32),
            pltpu.VMEM((H // 2, N), jnp.int32),
            pltpu.SemaphoreType.DMA,
            pltpu.SemaphoreType.DMA,
            pltpu.SemaphoreType.DMA,
            pltpu.SemaphoreType.DMA,
            pltpu.SemaphoreType.DMA,
            pltpu.SemaphoreType.DMA,
        ],
        compiler_params=pltpu.CompilerParams(
            use_tc_tiling_on_sc=False, needs_layout_passes=False),
    )(spat_tab_f32, edge_tab_f32, sp_idx, aet_idx)


def _tc_assemble_body(ab_ref, mid_ref, t_ref, out_ref):
    tv = t_ref[0]
    ab_row0 = ab_ref[0, 0, :] * 2.0
    out_ref[0, 0, :, :] = ab_row0[None, :] + tv[:, None]
    ab2 = ab_ref[0, 1:, :] * 2.0
    w = mid_ref[0]
    lo = lax.bitcast_convert_type(w << 16, jnp.float32)
    hi = lax.bitcast_convert_type(
        w & jnp.int32(-65536), jnp.float32)
    halves = jnp.concatenate([lo, hi], axis=1)
    tcol = jnp.broadcast_to(tv[None, :, None], (N, H, 1))
    bias = jnp.concatenate([tcol, halves], axis=2)
    out_ref[0, 1:, :, :] = ab2[:, None, :] + bias


def _tc_assemble(attn_bias, mid, t):
    return pl.pallas_call(
        _tc_assemble_body,
        grid=(B,),
        in_specs=[
            pl.BlockSpec((1, NP1, NP1), lambda b: (b, 0, 0)),
            pl.BlockSpec((1, N, H // 2, N), lambda b: (b, 0, 0, 0)),
            pl.BlockSpec((1, H), lambda b: (0, 0)),
        ],
        out_specs=pl.BlockSpec((1, NP1, H, NP1), lambda b: (b, 0, 0, 0)),
        out_shape=jax.ShapeDtypeStruct((B, NP1, H, NP1), jnp.float32),
    )(attn_bias, mid, t)


@jax.jit
def _run(attn_bias, spatial_pos, attn_edge_type,
         edge_encoder_weight, spatial_pos_encoder_weight,
         graph_token_virtual_distance_weight):
    def _pack_tab(tab_f32):
        tb = tab_f32.astype(jnp.bfloat16)
        lo = lax.bitcast_convert_type(tb[:, :16], jnp.uint16)
        hi = lax.bitcast_convert_type(tb[:, 16:], jnp.uint16)
        w = lo.astype(jnp.uint32) | (hi.astype(jnp.uint32) << 16)
        return lax.bitcast_convert_type(w, jnp.int32)

    spat_tab = _pack_tab(spatial_pos_encoder_weight)
    edge_tab = _pack_tab(edge_encoder_weight * (1.0 / 3.0))
    sp_idx = spatial_pos.reshape(NUM_ITEMS, N)
    aet_idx = attn_edge_type.reshape(NUM_ITEMS, 3 * N)
    mid = _sc_gather(spat_tab, edge_tab, sp_idx, aet_idx)
    out_phys = _tc_assemble(attn_bias, mid,
                            graph_token_virtual_distance_weight)
    return jnp.transpose(out_phys, (0, 2, 1, 3))


def kernel(attn_bias, spatial_pos, x, edge_input, attn_edge_type,
           edge_encoder_weight, spatial_pos_encoder_weight,
           graph_token_virtual_distance_weight):
    del x, edge_input
    return _run(attn_bias, spatial_pos, attn_edge_type,
                edge_encoder_weight, spatial_pos_encoder_weight,
                graph_token_virtual_distance_weight)
